# baseline (speedup 1.0000x reference)
import functools

import jax
import jax.numpy as jnp
from jax.experimental import pallas as pl


_EPS = 1e-12
_BIG = 2**30


def _rpq_body(x_ref, rp_ref, cbt_ref, out_ref):
    proj = jnp.dot(x_ref[...], rp_ref[...], preferred_element_type=jnp.float32)
    norm = jnp.sqrt(jnp.sum(proj * proj, axis=1, keepdims=True))
    projn = proj / jnp.maximum(norm, _EPS)
    x_sq = jnp.sum(projn * projn, axis=1, keepdims=True)

    cbt = cbt_ref[...]
    n = jnp.sqrt(jnp.sum(cbt * cbt, axis=0, keepdims=True))
    cbn = cbt / jnp.maximum(n, _EPS)
    cb_sq = jnp.sum(cbn * cbn, axis=0, keepdims=True)

    cross = jnp.dot(projn, cbn, preferred_element_type=jnp.float32)
    d2 = jnp.maximum(cb_sq + x_sq - 2.0 * cross, 0.0)
    dist = jnp.sqrt(d2)

    m = jnp.min(dist, axis=1, keepdims=True)
    iota = jax.lax.broadcasted_iota(jnp.int32, dist.shape, 1)
    idx = jnp.min(jnp.where(dist == m, iota, _BIG), axis=1)
    out_ref[0, 0, :] = idx.astype(jnp.int32)


@functools.partial(jax.jit, static_argnames=())
def _rpq(x2, rp, cbt):
    bn, d = x2.shape
    k = cbt.shape[1]
    block_rows = 512
    nb = bn // block_rows
    out = pl.pallas_call(
        _rpq_body,
        grid=(nb,),
        in_specs=[
            pl.BlockSpec((block_rows, d), lambda i: (i, 0)),
            pl.BlockSpec((d, rp.shape[1]), lambda i: (0, 0)),
            pl.BlockSpec((cbt.shape[0], k), lambda i: (0, 0)),
        ],
        out_specs=pl.BlockSpec((1, 1, block_rows), lambda i: (i, 0, 0)),
        out_shape=jax.ShapeDtypeStruct((nb, 1, block_rows), jnp.int32),
    )(x2, rp, cbt)
    return out.reshape(bn)


def kernel(x, random_projection, codebook):
    b, n, d = x.shape
    x2 = x.reshape(b * n, d)
    cbt = codebook.T
    idx = _rpq(x2, random_projection, cbt)
    return idx.reshape(b, n)


# speedup vs baseline: 1.2667x; 1.2667x over previous
import functools

import jax
import jax.numpy as jnp
from jax.experimental import pallas as pl


_EPS = 1e-12
_BIG = 2**30


def _rpq_body(x_ref, rp_ref, cbt_ref, out_ref):
    proj = jnp.dot(x_ref[...], rp_ref[...], preferred_element_type=jnp.float32)
    norm = jnp.sqrt(jnp.sum(proj * proj, axis=1, keepdims=True))
    projn = proj / jnp.maximum(norm, _EPS)
    x_sq = jnp.sum(projn * projn, axis=1, keepdims=True)

    cbt = cbt_ref[...]
    n = jnp.sqrt(jnp.sum(cbt * cbt, axis=0, keepdims=True))
    cbn = cbt / jnp.maximum(n, _EPS)
    cb_sq = jnp.sum(cbn * cbn, axis=0, keepdims=True)

    cm2 = jnp.dot(projn * (-2.0), cbn, preferred_element_type=jnp.float32)
    d2 = (cb_sq + x_sq) + cm2
    m2 = jnp.min(d2, axis=1, keepdims=True)

    s = jnp.sqrt(jnp.maximum(m2, 0.0))
    base = s * s
    bi = jax.lax.bitcast_convert_type(base, jnp.int32)
    x_hi = base
    for j in range(1, 7):
        xj = jax.lax.bitcast_convert_type(bi + j, jnp.float32)
        ok = jnp.sqrt(xj) == s
        x_hi = jnp.where(ok, xj, x_hi)

    iota = jax.lax.broadcasted_iota(jnp.int32, d2.shape, 1)
    idx = jnp.min(jnp.where(d2 <= x_hi, iota, _BIG), axis=1)
    out_ref[0, 0, :] = idx.astype(jnp.int32)


@functools.partial(jax.jit, static_argnames=())
def _rpq(x2, rp, cbt):
    bn, d = x2.shape
    k = cbt.shape[1]
    block_rows = 512
    nb = bn // block_rows
    out = pl.pallas_call(
        _rpq_body,
        grid=(nb,),
        in_specs=[
            pl.BlockSpec((block_rows, d), lambda i: (i, 0)),
            pl.BlockSpec((d, rp.shape[1]), lambda i: (0, 0)),
            pl.BlockSpec((cbt.shape[0], k), lambda i: (0, 0)),
        ],
        out_specs=pl.BlockSpec((1, 1, block_rows), lambda i: (i, 0, 0)),
        out_shape=jax.ShapeDtypeStruct((nb, 1, block_rows), jnp.int32),
    )(x2, rp, cbt)
    return out.reshape(bn)


def kernel(x, random_projection, codebook):
    b, n, d = x.shape
    x2 = x.reshape(b * n, d)
    cbt = codebook.T
    idx = _rpq(x2, random_projection, cbt)
    return idx.reshape(b, n)


# speedup vs baseline: 1.4465x; 1.1419x over previous
import functools

import jax
import jax.numpy as jnp
from jax.experimental import pallas as pl


_EPS = 1e-12
_BIG = 2**30


def _rpq_body(x_ref, rp_ref, cbt_ref, out_ref):
    proj = jnp.dot(x_ref[...], rp_ref[...], preferred_element_type=jnp.float32)
    norm = jnp.sqrt(jnp.sum(proj * proj, axis=1, keepdims=True))
    projn = proj / jnp.maximum(norm, _EPS)
    x_sq = jnp.sum(projn * projn, axis=1, keepdims=True)

    cbt = cbt_ref[...]
    n = jnp.sqrt(jnp.sum(cbt * cbt, axis=0, keepdims=True))
    cbn = cbt / jnp.maximum(n, _EPS)
    cb_sq = jnp.sum(cbn * cbn, axis=0, keepdims=True)

    cm2 = jnp.dot(projn * (-2.0), cbn, preferred_element_type=jnp.float32)
    d2 = (cb_sq + x_sq) + cm2
    m2 = jnp.min(d2, axis=1, keepdims=True)

    s = jnp.sqrt(jnp.maximum(m2, 0.0))
    base = s * s
    bi = jax.lax.bitcast_convert_type(base, jnp.int32)
    x_hi = base
    for j in range(1, 7):
        xj = jax.lax.bitcast_convert_type(bi + j, jnp.float32)
        ok = jnp.sqrt(xj) == s
        x_hi = jnp.where(ok, xj, x_hi)

    iota = jax.lax.broadcasted_iota(jnp.int32, d2.shape, 1).astype(jnp.float32)
    idx = jnp.min(jnp.where(d2 <= x_hi, iota, jnp.inf), axis=1)
    out_ref[0, 0, :] = idx.astype(jnp.int32)


@functools.partial(jax.jit, static_argnames=())
def _rpq(x2, rp, cbt):
    bn, d = x2.shape
    k = cbt.shape[1]
    block_rows = 512
    nb = bn // block_rows
    out = pl.pallas_call(
        _rpq_body,
        grid=(nb,),
        in_specs=[
            pl.BlockSpec((block_rows, d), lambda i: (i, 0)),
            pl.BlockSpec((d, rp.shape[1]), lambda i: (0, 0)),
            pl.BlockSpec((cbt.shape[0], k), lambda i: (0, 0)),
        ],
        out_specs=pl.BlockSpec((1, 1, block_rows), lambda i: (i, 0, 0)),
        out_shape=jax.ShapeDtypeStruct((nb, 1, block_rows), jnp.int32),
    )(x2, rp, cbt)
    return out.reshape(bn)


def kernel(x, random_projection, codebook):
    b, n, d = x.shape
    x2 = x.reshape(b * n, d)
    cbt = codebook.T
    idx = _rpq(x2, random_projection, cbt)
    return idx.reshape(b, n)


# speedup vs baseline: 1.4625x; 1.0111x over previous
import functools

import jax
import jax.numpy as jnp
from jax.experimental import pallas as pl


_EPS = 1e-12
_BIG = 2**30


def _rpq_body(x_ref, rp_ref, cbt_ref, iota_ref, out_ref):
    proj = jnp.dot(x_ref[...], rp_ref[...], preferred_element_type=jnp.float32)
    norm = jnp.sqrt(jnp.sum(proj * proj, axis=1, keepdims=True))
    projn = proj / jnp.maximum(norm, _EPS)
    x_sq = jnp.sum(projn * projn, axis=1, keepdims=True)

    cbt = cbt_ref[...]
    n = jnp.sqrt(jnp.sum(cbt * cbt, axis=0, keepdims=True))
    cbn = cbt / jnp.maximum(n, _EPS)
    cb_sq = jnp.sum(cbn * cbn, axis=0, keepdims=True)

    cm2 = jnp.dot(projn * (-2.0), cbn, preferred_element_type=jnp.float32)
    d2 = (cb_sq + x_sq) + cm2
    m2 = jnp.min(d2, axis=1, keepdims=True)

    s = jnp.sqrt(jnp.maximum(m2, 0.0))
    base = s * s
    bi = jax.lax.bitcast_convert_type(base, jnp.int32)
    x_hi = base
    for j in range(1, 7):
        xj = jax.lax.bitcast_convert_type(bi + j, jnp.float32)
        ok = jnp.sqrt(xj) == s
        x_hi = jnp.where(ok, xj, x_hi)

    idx = jnp.min(jnp.where(d2 <= x_hi, iota_ref[...], jnp.inf), axis=1)
    out_ref[0, 0, :] = idx.astype(jnp.int32)


@functools.partial(jax.jit, static_argnames=())
def _rpq(x2, rp, cbt):
    bn, d = x2.shape
    k = cbt.shape[1]
    block_rows = 512
    nb = bn // block_rows
    iota = jnp.arange(k, dtype=jnp.float32).reshape(1, k)
    out = pl.pallas_call(
        _rpq_body,
        grid=(nb,),
        in_specs=[
            pl.BlockSpec((block_rows, d), lambda i: (i, 0)),
            pl.BlockSpec((d, rp.shape[1]), lambda i: (0, 0)),
            pl.BlockSpec((cbt.shape[0], k), lambda i: (0, 0)),
            pl.BlockSpec((1, k), lambda i: (0, 0)),
        ],
        out_specs=pl.BlockSpec((1, 1, block_rows), lambda i: (i, 0, 0)),
        out_shape=jax.ShapeDtypeStruct((nb, 1, block_rows), jnp.int32),
    )(x2, rp, cbt, iota)
    return out.reshape(bn)


def kernel(x, random_projection, codebook):
    b, n, d = x.shape
    x2 = x.reshape(b * n, d)
    cbt = codebook.T
    idx = _rpq(x2, random_projection, cbt)
    return idx.reshape(b, n)


# speedup vs baseline: 1.4670x; 1.0031x over previous
import functools

import jax
import jax.numpy as jnp
from jax.experimental import pallas as pl


_EPS = 1e-12
_BIG = 2**30


def _rpq_body(x_ref, rp_ref, cbt_ref, iota_ref, out_ref):
    proj = jnp.dot(x_ref[...], rp_ref[...], preferred_element_type=jnp.float32)
    norm = jnp.sqrt(jnp.sum(proj * proj, axis=1, keepdims=True))
    projn = proj / jnp.maximum(norm, _EPS)
    x_sq = jnp.sum(projn * projn, axis=1, keepdims=True)

    cbt = cbt_ref[...]
    n = jnp.sqrt(jnp.sum(cbt * cbt, axis=0, keepdims=True))
    cbn = cbt / jnp.maximum(n, _EPS)
    cb_sq = jnp.sum(cbn * cbn, axis=0, keepdims=True)

    cm2 = jnp.dot(projn * (-2.0), cbn, preferred_element_type=jnp.float32)
    d2 = (cb_sq + x_sq) + cm2
    m2 = jnp.min(d2, axis=1, keepdims=True)

    s = jnp.sqrt(jnp.maximum(m2, 0.0))
    base = s * s
    bi = jax.lax.bitcast_convert_type(base, jnp.int32)
    x_hi = base
    for j in range(1, 7):
        xj = jax.lax.bitcast_convert_type(bi + j, jnp.float32)
        ok = jnp.sqrt(xj) == s
        x_hi = jnp.where(ok, xj, x_hi)

    idx = jnp.min(jnp.where(d2 <= x_hi, iota_ref[...], jnp.inf), axis=1)
    out_ref[0, 0, :] = idx.astype(jnp.int32)


@functools.partial(jax.jit, static_argnames=())
def _rpq(x2, rp, cbt):
    bn, d = x2.shape
    k = cbt.shape[1]
    block_rows = 1024
    nb = bn // block_rows
    iota = jnp.arange(k, dtype=jnp.float32).reshape(1, k)
    out = pl.pallas_call(
        _rpq_body,
        grid=(nb,),
        in_specs=[
            pl.BlockSpec((block_rows, d), lambda i: (i, 0)),
            pl.BlockSpec((d, rp.shape[1]), lambda i: (0, 0)),
            pl.BlockSpec((cbt.shape[0], k), lambda i: (0, 0)),
            pl.BlockSpec((1, k), lambda i: (0, 0)),
        ],
        out_specs=pl.BlockSpec((1, 1, block_rows), lambda i: (i, 0, 0)),
        out_shape=jax.ShapeDtypeStruct((nb, 1, block_rows), jnp.int32),
    )(x2, rp, cbt, iota)
    return out.reshape(bn)


def kernel(x, random_projection, codebook):
    b, n, d = x.shape
    x2 = x.reshape(b * n, d)
    cbt = codebook.T
    idx = _rpq(x2, random_projection, cbt)
    return idx.reshape(b, n)
